# Initial kernel scaffold; baseline (speedup 1.0000x reference)
import jax
import jax.numpy as jnp
from jax.experimental import pallas as pl


def kernel(hidden_preactivation_BH):
    raise NotImplementedError("write your pallas kernel here")


# speedup vs baseline: 3.9939x; 3.9939x over previous
import jax
import jax.numpy as jnp
from jax.experimental import pallas as pl

_TOPK = 64
_ROWS_PER_BLOCK = 8


def _sortable_key(x):
    u = jax.lax.bitcast_convert_type(x, jnp.int32)
    return jnp.where(u >= 0, u, jnp.int32(-2147483648) - u)


def _topk_mask_body(x_ref, o_ref):
    x = x_ref[...]
    key = _sortable_key(x)

    lo = jnp.full((_ROWS_PER_BLOCK, 1), -2**31, jnp.int32)
    hi = jnp.full((_ROWS_PER_BLOCK, 1), 2**31 - 1, jnp.int32)

    def body(_, lohi):
        lo, hi = lohi
        mid = (lo >> 1) + (hi >> 1) + (lo & hi & 1) + ((lo ^ hi) & 1)
        cnt = jnp.sum((key >= mid).astype(jnp.int32), axis=1, keepdims=True)
        p = cnt >= _TOPK
        return jnp.where(p, mid, lo), jnp.where(p, hi, mid - 1)

    lo, hi = jax.lax.fori_loop(0, 32, body, (lo, hi))
    t = lo

    gt = key > t
    eq = key == t
    c_gt = jnp.sum(gt.astype(jnp.int32), axis=1, keepdims=True)
    need = _TOPK - c_gt
    eq_i = eq.astype(jnp.int32)
    s = eq_i
    shift = 1
    h = x.shape[1]
    while shift < h:
        shifted = jnp.concatenate(
            [jnp.zeros((_ROWS_PER_BLOCK, shift), jnp.int32), s[:, :-shift]], axis=1)
        s = s + shifted
        shift *= 2
    eq_rank = s - eq_i
    take = gt | (eq & (eq_rank < need))
    o_ref[...] = jnp.where(take, x, jnp.float32(0.0))


def kernel(hidden_preactivation_BH):
    b, h = hidden_preactivation_BH.shape
    return pl.pallas_call(
        _topk_mask_body,
        grid=(b // _ROWS_PER_BLOCK,),
        in_specs=[pl.BlockSpec((_ROWS_PER_BLOCK, h), lambda i: (i, 0))],
        out_specs=pl.BlockSpec((_ROWS_PER_BLOCK, h), lambda i: (i, 0)),
        out_shape=jax.ShapeDtypeStruct((b, h), jnp.float32),
    )(hidden_preactivation_BH)
